# baseline (speedup 1.0000x reference)
import functools

import jax
import jax.numpy as jnp
import numpy as np
from jax import lax
from jax.experimental import pallas as pl
from jax.experimental.pallas import tpu as pltpu
from jax.experimental.pallas import tpu_sc as plsc

_CLASSES = 9
_CM1 = _CLASSES - 1
_INTR = np.array(
    [[572.4114, 0.0, 325.2611], [0.0, 573.57043, 242.04899], [0.0, 0.0, 1.0]],
    dtype=np.float32,
)
_KINV = np.linalg.inv(_INTR).astype(np.float32)

_B, _H, _W = 8, 224, 224
_HW = _H * _W
_ROWS = _HW // 128
_RT_H = 56

_ROWS_SC = 168
_ROWS_TC = _ROWS - _ROWS_SC
_NHT_TC = _ROWS_TC // _RT_H

_NSLOT = 11
_PS_ROWS = 96

_NW = 32
_WPB = _NW // _B
_PIX_SC_B = _ROWS_SC * 128
_BASE_SC = _ROWS_TC * 128
_PIX_W = _PIX_SC_B // _WPB
_P = 448
_NGRP = _P // 16
_NCHUNK = _PIX_W // _P


def _tc_gather_body(cat_ref, q_ref, s_ref, xy_ref, z_ref,
                    gq_ref, gs_ref, gxy_ref, gz_ref, ps_ref):
    h = pl.program_id(1)
    cm = cat_ref[0]
    idx = jnp.clip(cm - 1, 0, _CM1 - 1)
    fg = cm > 0

    @pl.when(h == 0)
    def _():
        ps_ref[...] = jnp.zeros((1, _PS_ROWS, 128), jnp.float32)

    fields = ((q_ref, gq_ref, 4, 0), (s_ref, gs_ref, 3, 4),
              (xy_ref, gxy_ref, 2, 7), (z_ref, None, 1, 9))

    for c in range(_CM1):
        m = jnp.where((idx == c) & fg, 1.0, 0.0)
        r = 10 * 8 + c
        ps_ref[0, pl.ds(r, 1), :] = ps_ref[0, pl.ds(r, 1), :] + jnp.sum(
            m, axis=0, keepdims=True)
        for in_ref, out_ref, nch, slot0 in fields:
            for ch in range(nch):
                p = m * in_ref[0, c * nch + ch]
                r = (slot0 + ch) * 8 + c
                ps_ref[0, pl.ds(r, 1), :] = ps_ref[0, pl.ds(r, 1), :] + jnp.sum(
                    p, axis=0, keepdims=True)
                if out_ref is None:
                    if c == 0:
                        gz_ref[0] = p
                    else:
                        gz_ref[0] = gz_ref[0] + p
                else:
                    if c == 0:
                        out_ref[0, ch] = p
                    else:
                        out_ref[0, ch] = out_ref[0, ch] + p


def _sc_gather_body(cm_hbm, q_hbm, s_hbm, xy_hbm, z_hbm,
                    gq_hbm, gs_hbm, gxy_hbm, gz_hbm, part_hbm,
                    cm_v, q_v, s_v, xy_v, z_v,
                    gq_v, gs_v, gxy_v, gz_v, acc_v,
                    in_sem0, in_sem1, out_sem0, out_sem1):
    in_sems = (in_sem0, in_sem1)
    out_sems = (out_sem0, out_sem1)
    wid = lax.axis_index("s") * 2 + lax.axis_index("c")
    b = wid // _WPB
    base = (wid % _WPB) * _PIX_W

    for sl in range(_NSLOT):
        for r in range(_CM1):
            acc_v[sl, r, :] = jnp.zeros((16,), jnp.float32)

    cols0 = lax.iota(jnp.int32, 16)
    ones = jnp.ones((16,), jnp.float32)

    def issue_in(off, k):
        src = _BASE_SC + off
        pltpu.async_copy(cm_hbm.at[b, pl.ds(src, _P)], cm_v.at[k], in_sems[k])
        pltpu.async_copy(q_hbm.at[b, :, pl.ds(src, _P)], q_v.at[k], in_sems[k])
        pltpu.async_copy(s_hbm.at[b, :, pl.ds(src, _P)], s_v.at[k], in_sems[k])
        pltpu.async_copy(xy_hbm.at[b, :, pl.ds(src, _P)], xy_v.at[k],
                         in_sems[k])
        pltpu.async_copy(z_hbm.at[b, :, pl.ds(src, _P)], z_v.at[k], in_sems[k])

    def drain_in(k):
        pltpu.make_async_copy(cm_hbm.at[0, pl.ds(0, _P)], cm_v.at[k],
                              in_sems[k]).wait()
        pltpu.make_async_copy(q_hbm.at[0, :, pl.ds(0, _P)], q_v.at[k],
                              in_sems[k]).wait()
        pltpu.make_async_copy(s_hbm.at[0, :, pl.ds(0, _P)], s_v.at[k],
                              in_sems[k]).wait()
        pltpu.make_async_copy(xy_hbm.at[0, :, pl.ds(0, _P)], xy_v.at[k],
                              in_sems[k]).wait()
        pltpu.make_async_copy(z_hbm.at[0, :, pl.ds(0, _P)], z_v.at[k],
                              in_sems[k]).wait()

    def issue_out(off, k):
        pltpu.async_copy(gq_v.at[k], gq_hbm.at[b, :, pl.ds(off, _P)],
                         out_sems[k])
        pltpu.async_copy(gs_v.at[k], gs_hbm.at[b, :, pl.ds(off, _P)],
                         out_sems[k])
        pltpu.async_copy(gxy_v.at[k], gxy_hbm.at[b, :, pl.ds(off, _P)],
                         out_sems[k])
        pltpu.async_copy(gz_v.at[k], gz_hbm.at[b, pl.ds(off, _P)],
                         out_sems[k])

    def drain_out(k):
        pltpu.make_async_copy(gq_v.at[k], gq_hbm.at[0, :, pl.ds(0, _P)],
                              out_sems[k]).wait()
        pltpu.make_async_copy(gs_v.at[k], gs_hbm.at[0, :, pl.ds(0, _P)],
                              out_sems[k]).wait()
        pltpu.make_async_copy(gxy_v.at[k], gxy_hbm.at[0, :, pl.ds(0, _P)],
                              out_sems[k]).wait()
        pltpu.make_async_copy(gz_v.at[k], gz_hbm.at[0, pl.ds(0, _P)],
                              out_sems[k]).wait()

    def compute(k):
        for g in range(_NGRP):
            cmv = cm_v[k, pl.ds(g * 16, 16)]
            idx = jnp.clip(cmv - 1, 0, _CM1 - 1)
            fg = cmv > 0
            cols = cols0 + g * 16
            plsc.addupdate_scatter(
                acc_v, [jnp.full((16,), 10, jnp.int32), idx, cols0],
                ones, mask=fg)
            for src, dst, nch, slot0 in ((q_v, gq_v, 4, 0), (s_v, gs_v, 3, 4),
                                         (xy_v, gxy_v, 2, 7)):
                for ch in range(nch):
                    v = plsc.load_gather(src.at[k], [idx * nch + ch, cols])
                    v = jnp.where(fg, v, 0.0)
                    dst[k, ch, pl.ds(g * 16, 16)] = v
                    plsc.addupdate_scatter(
                        acc_v,
                        [jnp.full((16,), slot0 + ch, jnp.int32), idx, cols0],
                        v, mask=fg)
            v = plsc.load_gather(z_v.at[k], [idx, cols])
            v = jnp.where(fg, v, 0.0)
            gz_v[k, pl.ds(g * 16, 16)] = v
            plsc.addupdate_scatter(
                acc_v, [jnp.full((16,), 9, jnp.int32), idx, cols0],
                v, mask=fg)

    issue_in(base, 0)

    def pair(i, carry):
        offa = base + (2 * i) * _P
        offb = offa + _P
        issue_in(offb, 1)
        drain_in(0)

        @pl.when(i > 0)
        def _():
            drain_out(0)

        compute(0)
        issue_out(offa, 0)

        @pl.when(i < _NCHUNK // 2 - 1)
        def _():
            issue_in(offb + _P, 0)

        drain_in(1)

        @pl.when(i > 0)
        def _():
            drain_out(1)

        compute(1)
        issue_out(offb, 1)
        return carry

    lax.fori_loop(0, _NCHUNK // 2, pair, 0)
    drain_out(0)
    drain_out(1)
    pltpu.sync_copy(acc_v, part_hbm.at[wid])


def _epilogue_body(ps_ref, part_ref, out_ref):
    S_tc = jnp.sum(ps_ref[...], axis=2)
    S_sc = jnp.sum(part_ref[...], axis=2)
    n = _NSLOT * _CM1
    SS = (S_tc[:, 0:n] + S_sc[:, 0:n] + S_sc[:, n:2 * n]
          + S_sc[:, 2 * n:3 * n] + S_sc[:, 3 * n:4 * n])

    def sl(k):
        return SS[:, k * _CM1:(k + 1) * _CM1]

    cnt = sl(10)
    denom = jnp.maximum(cnt, 1.0)
    q0 = sl(0) / denom
    q1 = sl(1) / denom
    q2 = sl(2) / denom
    q3 = sl(3) / denom
    s0 = sl(4) / denom
    s1 = sl(5) / denom
    s2 = sl(6) / denom
    x0 = sl(7) / denom
    x1 = sl(8) / denom
    zm = sl(9) / denom
    nrm = jnp.maximum(jnp.sqrt(q0 * q0 + q1 * q1 + q2 * q2 + q3 * q3), 1e-8)
    qw, qx, qy, qz = q0 / nrm, q1 / nrm, q2 / nrm, q3 / nrm
    r00 = 1 - 2 * (qy * qy + qz * qz)
    r01 = 2 * (qx * qy - qz * qw)
    r02 = 2 * (qx * qz + qy * qw)
    r10 = 2 * (qx * qy + qz * qw)
    r11 = 1 - 2 * (qx * qx + qz * qz)
    r12 = 2 * (qy * qz - qx * qw)
    r20 = 2 * (qx * qz - qy * qw)
    r21 = 2 * (qy * qz + qx * qw)
    r22 = 1 - 2 * (qx * qx + qy * qy)
    zval = jnp.exp(zm)
    t0 = zval * (x0 * _KINV[0, 0] + x1 * _KINV[0, 1] + _KINV[0, 2])
    t1 = zval * (x0 * _KINV[1, 0] + x1 * _KINV[1, 1] + _KINV[1, 2])
    t2 = zval * (x0 * _KINV[2, 0] + x1 * _KINV[2, 1] + _KINV[2, 2])
    one = jnp.ones_like(q0)
    zero = jnp.zeros_like(q0)
    rows = [q0, q1, q2, q3, s0, s1, s2, x0, x1, zm, cnt,
            r00, r01, r02, t0, r10, r11, r12, t1, r20, r21, r22, t2,
            zero, zero, zero, one,
            zero, zero, zero, zero, zero]
    out_ref[...] = jnp.stack(rows, axis=0)


@functools.partial(jax.jit, static_argnums=())
def kernel(cat_mask, quaternion, scales, xy, z):
    B = cat_mask.shape[0]
    cm_f = cat_mask.reshape(B, _HW).astype(jnp.int32)
    q_f = quaternion.reshape(B, 4 * _CM1, _HW)
    s_f = scales.reshape(B, 3 * _CM1, _HW)
    x_f = xy.reshape(B, 2 * _CM1, _HW)
    z_f = z.reshape(B, _CM1, _HW)

    sc_fn = pl.kernel(
        _sc_gather_body,
        mesh=plsc.VectorSubcoreMesh(core_axis_name="c", subcore_axis_name="s"),
        compiler_params=pltpu.CompilerParams(
            use_tc_tiling_on_sc=False, needs_layout_passes=False),
        out_type=[
            jax.ShapeDtypeStruct((B, 4, _PIX_SC_B), jnp.float32),
            jax.ShapeDtypeStruct((B, 3, _PIX_SC_B), jnp.float32),
            jax.ShapeDtypeStruct((B, 2, _PIX_SC_B), jnp.float32),
            jax.ShapeDtypeStruct((B, _PIX_SC_B), jnp.float32),
            jax.ShapeDtypeStruct((_NW, _NSLOT, _CM1, 16), jnp.float32),
        ],
        scratch_types=[
            pltpu.VMEM((2, _P), jnp.int32),
            pltpu.VMEM((2, 4 * _CM1, _P), jnp.float32),
            pltpu.VMEM((2, 3 * _CM1, _P), jnp.float32),
            pltpu.VMEM((2, 2 * _CM1, _P), jnp.float32),
            pltpu.VMEM((2, _CM1, _P), jnp.float32),
            pltpu.VMEM((2, 4, _P), jnp.float32),
            pltpu.VMEM((2, 3, _P), jnp.float32),
            pltpu.VMEM((2, 2, _P), jnp.float32),
            pltpu.VMEM((2, _P), jnp.float32),
            pltpu.VMEM((_NSLOT, _CM1, 16), jnp.float32),
            pltpu.SemaphoreType.DMA,
            pltpu.SemaphoreType.DMA,
            pltpu.SemaphoreType.DMA,
            pltpu.SemaphoreType.DMA,
        ],
    )
    gq_sc, gs_sc, gxy_sc, gz_sc, part = sc_fn(cm_f, q_f, s_f, x_f, z_f)

    cm4 = cm_f.reshape(B, _ROWS, 128)
    q4 = q_f.reshape(B, 4 * _CM1, _ROWS, 128)
    s4 = s_f.reshape(B, 3 * _CM1, _ROWS, 128)
    x4 = x_f.reshape(B, 2 * _CM1, _ROWS, 128)
    z4 = z_f.reshape(B, _CM1, _ROWS, 128)
    grid = (B, _NHT_TC)
    gq_tc, gs_tc, gxy_tc, gz_tc, psums = pl.pallas_call(
        _tc_gather_body,
        grid=grid,
        in_specs=[
            pl.BlockSpec((1, _RT_H, 128), lambda b, h: (b, h, 0)),
            pl.BlockSpec((1, 4 * _CM1, _RT_H, 128), lambda b, h: (b, 0, h, 0)),
            pl.BlockSpec((1, 3 * _CM1, _RT_H, 128), lambda b, h: (b, 0, h, 0)),
            pl.BlockSpec((1, 2 * _CM1, _RT_H, 128), lambda b, h: (b, 0, h, 0)),
            pl.BlockSpec((1, _CM1, _RT_H, 128), lambda b, h: (b, 0, h, 0)),
        ],
        out_specs=(
            pl.BlockSpec((1, 4, _RT_H, 128), lambda b, h: (b, 0, h, 0)),
            pl.BlockSpec((1, 3, _RT_H, 128), lambda b, h: (b, 0, h, 0)),
            pl.BlockSpec((1, 2, _RT_H, 128), lambda b, h: (b, 0, h, 0)),
            pl.BlockSpec((1, _RT_H, 128), lambda b, h: (b, h, 0)),
            pl.BlockSpec((1, _PS_ROWS, 128), lambda b, h: (b, 0, 0)),
        ),
        out_shape=(
            jax.ShapeDtypeStruct((B, 4, _ROWS_TC, 128), jnp.float32),
            jax.ShapeDtypeStruct((B, 3, _ROWS_TC, 128), jnp.float32),
            jax.ShapeDtypeStruct((B, 2, _ROWS_TC, 128), jnp.float32),
            jax.ShapeDtypeStruct((B, _ROWS_TC, 128), jnp.float32),
            jax.ShapeDtypeStruct((B, _PS_ROWS, 128), jnp.float32),
        ),
    )(cm4, q4, s4, x4, z4)

    E = pl.pallas_call(
        _epilogue_body,
        out_shape=jax.ShapeDtypeStruct((32, 8, 8), jnp.float32),
    )(psums, part.reshape(B, _WPB * _NSLOT * _CM1, 16))

    def col(r):
        return E[r].T.reshape(_CM1 * B)

    aq = jnp.stack([col(0), col(1), col(2), col(3)], axis=1)
    ascl = jnp.stack([col(4), col(5), col(6)], axis=1)
    axy = jnp.stack([col(7), col(8)], axis=1)
    az = col(9)[:, None]
    fg_counts = col(10)[:, None]
    RT = jnp.stack([col(11 + i) for i in range(16)], axis=1).reshape(
        _CM1 * B, 4, 4)

    def merge(tc_part, sc_part, nch):
        flat_tc = tc_part.reshape(B, nch, _ROWS_TC * 128)
        return jnp.concatenate([flat_tc, sc_part], axis=2).reshape(
            B, nch, _H, _W)

    gq = merge(gq_tc, gq_sc, 4)
    gs = merge(gs_tc, gs_sc, 3)
    gxy = merge(gxy_tc, gxy_sc, 2)
    gz = jnp.concatenate(
        [gz_tc.reshape(B, _ROWS_TC * 128), gz_sc], axis=1).reshape(B, _H, _W)
    return aq, ascl, axy, az, RT, fg_counts, gq, gs, gxy, gz


# speedup vs baseline: 1.8854x; 1.8854x over previous
import functools

import jax
import jax.numpy as jnp
import numpy as np
from jax import lax
from jax.experimental import pallas as pl
from jax.experimental.pallas import tpu as pltpu
from jax.experimental.pallas import tpu_sc as plsc

_CLASSES = 9
_CM1 = _CLASSES - 1
_INTR = np.array(
    [[572.4114, 0.0, 325.2611], [0.0, 573.57043, 242.04899], [0.0, 0.0, 1.0]],
    dtype=np.float32,
)
_KINV = np.linalg.inv(_INTR).astype(np.float32)

_B, _H, _W = 8, 224, 224
_HW = _H * _W
_ROWS = _HW // 128
_RT_H = 392

_ROWS_SC = 0
_ROWS_TC = _ROWS - _ROWS_SC
_NHT_TC = _ROWS_TC // _RT_H

_NSLOT = 11
_PS_ROWS = 96

_NW = 32
_WPB = _NW // _B
_PIX_SC_B = _ROWS_SC * 128
_BASE_SC = _ROWS_TC * 128
_PIX_W = _PIX_SC_B // _WPB
_P = 448
_NGRP = _P // 16
_NCHUNK = _PIX_W // _P


def _tc_gather_body(cat_ref, q_ref, s_ref, xy_ref, z_ref,
                    gq_ref, gs_ref, gxy_ref, gz_ref, ps_ref):
    h = pl.program_id(1)
    cm = cat_ref[0]
    idx = jnp.clip(cm - 1, 0, _CM1 - 1)
    fg = cm > 0

    @pl.when(h == 0)
    def _():
        ps_ref[...] = jnp.zeros((1, _PS_ROWS, 128), jnp.float32)

    fields = ((q_ref, gq_ref, 4, 0), (s_ref, gs_ref, 3, 4),
              (xy_ref, gxy_ref, 2, 7), (z_ref, None, 1, 9))

    for c in range(_CM1):
        m = jnp.where((idx == c) & fg, 1.0, 0.0)
        r = 10 * 8 + c
        ps_ref[0, pl.ds(r, 1), :] = ps_ref[0, pl.ds(r, 1), :] + jnp.sum(
            m, axis=0, keepdims=True)
        for in_ref, out_ref, nch, slot0 in fields:
            for ch in range(nch):
                p = m * in_ref[0, c * nch + ch]
                r = (slot0 + ch) * 8 + c
                ps_ref[0, pl.ds(r, 1), :] = ps_ref[0, pl.ds(r, 1), :] + jnp.sum(
                    p, axis=0, keepdims=True)
                if out_ref is None:
                    if c == 0:
                        gz_ref[0] = p
                    else:
                        gz_ref[0] = gz_ref[0] + p
                else:
                    if c == 0:
                        out_ref[0, ch] = p
                    else:
                        out_ref[0, ch] = out_ref[0, ch] + p


def _sc_gather_body(cm_hbm, q_hbm, s_hbm, xy_hbm, z_hbm,
                    gq_hbm, gs_hbm, gxy_hbm, gz_hbm, part_hbm,
                    cm_v, q_v, s_v, xy_v, z_v,
                    gq_v, gs_v, gxy_v, gz_v, acc_v,
                    in_sem0, in_sem1, out_sem0, out_sem1):
    in_sems = (in_sem0, in_sem1)
    out_sems = (out_sem0, out_sem1)
    wid = lax.axis_index("s") * 2 + lax.axis_index("c")
    b = wid // _WPB
    base = (wid % _WPB) * _PIX_W

    for sl in range(_NSLOT):
        for r in range(_CM1):
            acc_v[sl, r, :] = jnp.zeros((16,), jnp.float32)

    cols0 = lax.iota(jnp.int32, 16)
    ones = jnp.ones((16,), jnp.float32)

    def issue_in(off, k):
        src = _BASE_SC + off
        pltpu.async_copy(cm_hbm.at[b, pl.ds(src, _P)], cm_v.at[k], in_sems[k])
        pltpu.async_copy(q_hbm.at[b, :, pl.ds(src, _P)], q_v.at[k], in_sems[k])
        pltpu.async_copy(s_hbm.at[b, :, pl.ds(src, _P)], s_v.at[k], in_sems[k])
        pltpu.async_copy(xy_hbm.at[b, :, pl.ds(src, _P)], xy_v.at[k],
                         in_sems[k])
        pltpu.async_copy(z_hbm.at[b, :, pl.ds(src, _P)], z_v.at[k], in_sems[k])

    def drain_in(k):
        pltpu.make_async_copy(cm_hbm.at[0, pl.ds(0, _P)], cm_v.at[k],
                              in_sems[k]).wait()
        pltpu.make_async_copy(q_hbm.at[0, :, pl.ds(0, _P)], q_v.at[k],
                              in_sems[k]).wait()
        pltpu.make_async_copy(s_hbm.at[0, :, pl.ds(0, _P)], s_v.at[k],
                              in_sems[k]).wait()
        pltpu.make_async_copy(xy_hbm.at[0, :, pl.ds(0, _P)], xy_v.at[k],
                              in_sems[k]).wait()
        pltpu.make_async_copy(z_hbm.at[0, :, pl.ds(0, _P)], z_v.at[k],
                              in_sems[k]).wait()

    def issue_out(off, k):
        pltpu.async_copy(gq_v.at[k], gq_hbm.at[b, :, pl.ds(off, _P)],
                         out_sems[k])
        pltpu.async_copy(gs_v.at[k], gs_hbm.at[b, :, pl.ds(off, _P)],
                         out_sems[k])
        pltpu.async_copy(gxy_v.at[k], gxy_hbm.at[b, :, pl.ds(off, _P)],
                         out_sems[k])
        pltpu.async_copy(gz_v.at[k], gz_hbm.at[b, pl.ds(off, _P)],
                         out_sems[k])

    def drain_out(k):
        pltpu.make_async_copy(gq_v.at[k], gq_hbm.at[0, :, pl.ds(0, _P)],
                              out_sems[k]).wait()
        pltpu.make_async_copy(gs_v.at[k], gs_hbm.at[0, :, pl.ds(0, _P)],
                              out_sems[k]).wait()
        pltpu.make_async_copy(gxy_v.at[k], gxy_hbm.at[0, :, pl.ds(0, _P)],
                              out_sems[k]).wait()
        pltpu.make_async_copy(gz_v.at[k], gz_hbm.at[0, pl.ds(0, _P)],
                              out_sems[k]).wait()

    def compute(k):
        for g in range(_NGRP):
            cmv = cm_v[k, pl.ds(g * 16, 16)]
            idx = jnp.clip(cmv - 1, 0, _CM1 - 1)
            fg = cmv > 0
            cols = cols0 + g * 16
            plsc.addupdate_scatter(
                acc_v, [jnp.full((16,), 10, jnp.int32), idx, cols0],
                ones, mask=fg)
            for src, dst, nch, slot0 in ((q_v, gq_v, 4, 0), (s_v, gs_v, 3, 4),
                                         (xy_v, gxy_v, 2, 7)):
                for ch in range(nch):
                    v = plsc.load_gather(src.at[k], [idx * nch + ch, cols])
                    v = jnp.where(fg, v, 0.0)
                    dst[k, ch, pl.ds(g * 16, 16)] = v
                    plsc.addupdate_scatter(
                        acc_v,
                        [jnp.full((16,), slot0 + ch, jnp.int32), idx, cols0],
                        v, mask=fg)
            v = plsc.load_gather(z_v.at[k], [idx, cols])
            v = jnp.where(fg, v, 0.0)
            gz_v[k, pl.ds(g * 16, 16)] = v
            plsc.addupdate_scatter(
                acc_v, [jnp.full((16,), 9, jnp.int32), idx, cols0],
                v, mask=fg)

    issue_in(base, 0)

    def pair(i, carry):
        offa = base + (2 * i) * _P
        offb = offa + _P
        issue_in(offb, 1)
        drain_in(0)

        @pl.when(i > 0)
        def _():
            drain_out(0)

        compute(0)
        issue_out(offa, 0)

        @pl.when(i < _NCHUNK // 2 - 1)
        def _():
            issue_in(offb + _P, 0)

        drain_in(1)

        @pl.when(i > 0)
        def _():
            drain_out(1)

        compute(1)
        issue_out(offb, 1)
        return carry

    lax.fori_loop(0, _NCHUNK // 2, pair, 0)
    drain_out(0)
    drain_out(1)
    pltpu.sync_copy(acc_v, part_hbm.at[wid])


def _epilogue_body(ps_ref, part_ref, out_ref):
    S_tc = jnp.sum(ps_ref[...], axis=2)
    S_sc = jnp.sum(part_ref[...], axis=2)
    n = _NSLOT * _CM1
    SS = (S_tc[:, 0:n] + S_sc[:, 0:n] + S_sc[:, n:2 * n]
          + S_sc[:, 2 * n:3 * n] + S_sc[:, 3 * n:4 * n])

    def sl(k):
        return SS[:, k * _CM1:(k + 1) * _CM1]

    cnt = sl(10)
    denom = jnp.maximum(cnt, 1.0)
    q0 = sl(0) / denom
    q1 = sl(1) / denom
    q2 = sl(2) / denom
    q3 = sl(3) / denom
    s0 = sl(4) / denom
    s1 = sl(5) / denom
    s2 = sl(6) / denom
    x0 = sl(7) / denom
    x1 = sl(8) / denom
    zm = sl(9) / denom
    nrm = jnp.maximum(jnp.sqrt(q0 * q0 + q1 * q1 + q2 * q2 + q3 * q3), 1e-8)
    qw, qx, qy, qz = q0 / nrm, q1 / nrm, q2 / nrm, q3 / nrm
    r00 = 1 - 2 * (qy * qy + qz * qz)
    r01 = 2 * (qx * qy - qz * qw)
    r02 = 2 * (qx * qz + qy * qw)
    r10 = 2 * (qx * qy + qz * qw)
    r11 = 1 - 2 * (qx * qx + qz * qz)
    r12 = 2 * (qy * qz - qx * qw)
    r20 = 2 * (qx * qz - qy * qw)
    r21 = 2 * (qy * qz + qx * qw)
    r22 = 1 - 2 * (qx * qx + qy * qy)
    zval = jnp.exp(zm)
    t0 = zval * (x0 * _KINV[0, 0] + x1 * _KINV[0, 1] + _KINV[0, 2])
    t1 = zval * (x0 * _KINV[1, 0] + x1 * _KINV[1, 1] + _KINV[1, 2])
    t2 = zval * (x0 * _KINV[2, 0] + x1 * _KINV[2, 1] + _KINV[2, 2])
    one = jnp.ones_like(q0)
    zero = jnp.zeros_like(q0)
    rows = [q0, q1, q2, q3, s0, s1, s2, x0, x1, zm, cnt,
            r00, r01, r02, t0, r10, r11, r12, t1, r20, r21, r22, t2,
            zero, zero, zero, one,
            zero, zero, zero, zero, zero]
    out_ref[...] = jnp.stack(rows, axis=0)


@functools.partial(jax.jit, static_argnums=())
def kernel(cat_mask, quaternion, scales, xy, z):
    B = cat_mask.shape[0]
    cm_f = cat_mask.reshape(B, _HW).astype(jnp.int32)
    q_f = quaternion.reshape(B, 4 * _CM1, _HW)
    s_f = scales.reshape(B, 3 * _CM1, _HW)
    x_f = xy.reshape(B, 2 * _CM1, _HW)
    z_f = z.reshape(B, _CM1, _HW)

    sc_fn = pl.kernel(
        _sc_gather_body,
        mesh=plsc.VectorSubcoreMesh(core_axis_name="c", subcore_axis_name="s"),
        compiler_params=pltpu.CompilerParams(
            use_tc_tiling_on_sc=False, needs_layout_passes=False),
        out_type=[
            jax.ShapeDtypeStruct((B, 4, _PIX_SC_B), jnp.float32),
            jax.ShapeDtypeStruct((B, 3, _PIX_SC_B), jnp.float32),
            jax.ShapeDtypeStruct((B, 2, _PIX_SC_B), jnp.float32),
            jax.ShapeDtypeStruct((B, _PIX_SC_B), jnp.float32),
            jax.ShapeDtypeStruct((_NW, _NSLOT, _CM1, 16), jnp.float32),
        ],
        scratch_types=[
            pltpu.VMEM((2, _P), jnp.int32),
            pltpu.VMEM((2, 4 * _CM1, _P), jnp.float32),
            pltpu.VMEM((2, 3 * _CM1, _P), jnp.float32),
            pltpu.VMEM((2, 2 * _CM1, _P), jnp.float32),
            pltpu.VMEM((2, _CM1, _P), jnp.float32),
            pltpu.VMEM((2, 4, _P), jnp.float32),
            pltpu.VMEM((2, 3, _P), jnp.float32),
            pltpu.VMEM((2, 2, _P), jnp.float32),
            pltpu.VMEM((2, _P), jnp.float32),
            pltpu.VMEM((_NSLOT, _CM1, 16), jnp.float32),
            pltpu.SemaphoreType.DMA,
            pltpu.SemaphoreType.DMA,
            pltpu.SemaphoreType.DMA,
            pltpu.SemaphoreType.DMA,
        ],
    )


    cm4 = cm_f.reshape(B, _ROWS, 128)
    q4 = q_f.reshape(B, 4 * _CM1, _ROWS, 128)
    s4 = s_f.reshape(B, 3 * _CM1, _ROWS, 128)
    x4 = x_f.reshape(B, 2 * _CM1, _ROWS, 128)
    z4 = z_f.reshape(B, _CM1, _ROWS, 128)
    grid = (B, _NHT_TC)
    gq_tc, gs_tc, gxy_tc, gz_tc, psums = pl.pallas_call(
        _tc_gather_body,
        grid=grid,
        in_specs=[
            pl.BlockSpec((1, _RT_H, 128), lambda b, h: (b, h, 0)),
            pl.BlockSpec((1, 4 * _CM1, _RT_H, 128), lambda b, h: (b, 0, h, 0)),
            pl.BlockSpec((1, 3 * _CM1, _RT_H, 128), lambda b, h: (b, 0, h, 0)),
            pl.BlockSpec((1, 2 * _CM1, _RT_H, 128), lambda b, h: (b, 0, h, 0)),
            pl.BlockSpec((1, _CM1, _RT_H, 128), lambda b, h: (b, 0, h, 0)),
        ],
        out_specs=(
            pl.BlockSpec((1, 4, _RT_H, 128), lambda b, h: (b, 0, h, 0)),
            pl.BlockSpec((1, 3, _RT_H, 128), lambda b, h: (b, 0, h, 0)),
            pl.BlockSpec((1, 2, _RT_H, 128), lambda b, h: (b, 0, h, 0)),
            pl.BlockSpec((1, _RT_H, 128), lambda b, h: (b, h, 0)),
            pl.BlockSpec((1, _PS_ROWS, 128), lambda b, h: (b, 0, 0)),
        ),
        out_shape=(
            jax.ShapeDtypeStruct((B, 4, _ROWS_TC, 128), jnp.float32),
            jax.ShapeDtypeStruct((B, 3, _ROWS_TC, 128), jnp.float32),
            jax.ShapeDtypeStruct((B, 2, _ROWS_TC, 128), jnp.float32),
            jax.ShapeDtypeStruct((B, _ROWS_TC, 128), jnp.float32),
            jax.ShapeDtypeStruct((B, _PS_ROWS, 128), jnp.float32),
        ),
    )(cm4, q4, s4, x4, z4)

    E = pl.pallas_call(
        _epilogue_body,
        out_shape=jax.ShapeDtypeStruct((32, 8, 8), jnp.float32),
    )(psums, jnp.zeros((B, _WPB * _NSLOT * _CM1, 16), jnp.float32))

    def col(r):
        return E[r].T.reshape(_CM1 * B)

    aq = jnp.stack([col(0), col(1), col(2), col(3)], axis=1)
    ascl = jnp.stack([col(4), col(5), col(6)], axis=1)
    axy = jnp.stack([col(7), col(8)], axis=1)
    az = col(9)[:, None]
    fg_counts = col(10)[:, None]
    RT = jnp.stack([col(11 + i) for i in range(16)], axis=1).reshape(
        _CM1 * B, 4, 4)

    gq = gq_tc.reshape(B, 4, _H, _W)
    gs = gs_tc.reshape(B, 3, _H, _W)
    gxy = gxy_tc.reshape(B, 2, _H, _W)
    gz = gz_tc.reshape(B, _H, _W)
    return aq, ascl, axy, az, RT, fg_counts, gq, gs, gxy, gz
